# baseline (speedup 1.0000x reference)
import jax
import jax.numpy as jnp
from jax.experimental import pallas as pl
from jax.experimental.pallas import tpu as pltpu

N = 65536
K = 256
M = 64
N2 = N // 2
K2 = 2 * K
M2 = 2 * M
BLOCK = 4096
NBUF = 4
NSTEPS = N2 // BLOCK


def _mm_body(x_hbm, wt_ref, b_ref, o_hbm, *rest):
    xbufs = rest[:NBUF]
    obufs = rest[NBUF : 2 * NBUF]
    insems, outsems = rest[2 * NBUF], rest[2 * NBUF + 1]

    def in_copy(i, s):
        return pltpu.make_async_copy(
            x_hbm.at[pl.ds(i * BLOCK, BLOCK), :], xbufs[s], insems.at[s]
        )

    def out_copy(i, s):
        return pltpu.make_async_copy(
            obufs[s], o_hbm.at[pl.ds(i * BLOCK, BLOCK), :], outsems.at[s]
        )

    for i in range(NBUF):
        in_copy(i, i).start()
    for i in range(NSTEPS):
        s = i % NBUF
        in_copy(i, s).wait()
        if i >= NBUF:
            out_copy(i - NBUF, s).wait()
        x = xbufs[s][...]
        res_e = jnp.dot(x[:, :K], wt_ref[...], preferred_element_type=jnp.float32)
        res_o = jnp.dot(x[:, K:], wt_ref[...], preferred_element_type=jnp.float32)
        obufs[s][...] = jnp.concatenate([res_e, res_o], axis=1) + b_ref[...]
        out_copy(i, s).start()
        if i + NBUF < NSTEPS:
            in_copy(i + NBUF, s).start()
    for i in range(NSTEPS - NBUF, NSTEPS):
        out_copy(i, i % NBUF).wait()


@jax.jit
def _matmul(x2, wt, bias2):
    return pl.pallas_call(
        _mm_body,
        in_specs=[
            pl.BlockSpec(memory_space=pl.ANY),
            pl.BlockSpec(memory_space=pltpu.VMEM),
            pl.BlockSpec(memory_space=pltpu.VMEM),
        ],
        out_specs=pl.BlockSpec(memory_space=pl.ANY),
        out_shape=jax.ShapeDtypeStruct((N2, M2), jnp.float32),
        scratch_shapes=(
            [pltpu.VMEM((BLOCK, K2), jnp.float32) for _ in range(NBUF)]
            + [pltpu.VMEM((BLOCK, M2), jnp.float32) for _ in range(NBUF)]
            + [
                pltpu.SemaphoreType.DMA((NBUF,)),
                pltpu.SemaphoreType.DMA((NBUF,)),
            ]
        ),
    )(x2, wt, bias2)


def kernel(input, weight, bias):
    x2 = input.reshape(N2, K2)
    bias2 = jnp.tile(bias, 2).reshape(1, M2)
    out2 = _matmul(x2, weight.T, bias2)
    return out2.reshape(N, M)


# speedup vs baseline: 1.8098x; 1.8098x over previous
import jax
import jax.numpy as jnp
from jax.experimental import pallas as pl
from jax.experimental.pallas import tpu as pltpu

N = 65536
K = 256
M = 64
N2 = N // 2
M2 = 2 * M
BLOCK = 8192
NBUF = 4
NSTEPS = N // BLOCK


def _mm_body(x_hbm, wt_ref, b_ref, o_hbm, *rest):
    xbufs = rest[:NBUF]
    obufs = rest[NBUF : 2 * NBUF]
    insems, outsems = rest[2 * NBUF], rest[2 * NBUF + 1]

    def in_copy(i, s):
        return pltpu.make_async_copy(
            x_hbm.at[pl.ds(i * BLOCK, BLOCK), :], xbufs[s], insems.at[s]
        )

    def out_copy(i, s):
        return pltpu.make_async_copy(
            obufs[s],
            o_hbm.at[pl.ds(i * (BLOCK // 2), BLOCK // 2), :],
            outsems.at[s],
        )

    for i in range(NBUF):
        in_copy(i, i).start()
    for i in range(NSTEPS):
        s = i % NBUF
        in_copy(i, s).wait()
        if i >= NBUF:
            out_copy(i - NBUF, s).wait()
        res = (
            jnp.dot(
                xbufs[s][...], wt_ref[...], preferred_element_type=jnp.float32
            )
            + b_ref[...]
        )
        wide = jnp.concatenate([res, pltpu.roll(res, BLOCK - 1, 0)], axis=1)
        obufs[s][...] = wide.reshape(BLOCK // 2, 2, M2)[:, 0, :]
        out_copy(i, s).start()
        if i + NBUF < NSTEPS:
            in_copy(i + NBUF, s).start()
    for i in range(NSTEPS - NBUF, NSTEPS):
        out_copy(i, i % NBUF).wait()


@jax.jit
def _matmul(x, wt, bias2):
    return pl.pallas_call(
        _mm_body,
        in_specs=[
            pl.BlockSpec(memory_space=pl.ANY),
            pl.BlockSpec(memory_space=pltpu.VMEM),
            pl.BlockSpec(memory_space=pltpu.VMEM),
        ],
        out_specs=pl.BlockSpec(memory_space=pl.ANY),
        out_shape=jax.ShapeDtypeStruct((N2, M2), jnp.float32),
        scratch_shapes=(
            [pltpu.VMEM((BLOCK, K), jnp.float32) for _ in range(NBUF)]
            + [pltpu.VMEM((BLOCK // 2, M2), jnp.float32) for _ in range(NBUF)]
            + [
                pltpu.SemaphoreType.DMA((NBUF,)),
                pltpu.SemaphoreType.DMA((NBUF,)),
            ]
        ),
    )(x, wt, bias2)


def kernel(input, weight, bias):
    bias2 = bias.reshape(1, M)
    out2 = _matmul(input, weight.T, bias2)
    return out2.reshape(N, M)
